# Initial kernel scaffold; baseline (speedup 1.0000x reference)
import jax
import jax.numpy as jnp
from jax.experimental import pallas as pl


def kernel(x, edge_index, Wt, bt, W0, W1, W2):
    raise NotImplementedError("write your pallas kernel here")


# speedup vs baseline: 13.0663x; 13.0663x over previous
import functools

import jax
import jax.numpy as jnp
from jax import lax
from jax.experimental import pallas as pl
from jax.experimental.pallas import tpu as pltpu
from jax.experimental.pallas import tpu_sc as plsc

N = 10000
D = 128
E = 320000
NC = 2
NS = 16
NW = NC * NS
EPW = E // NW
C = 80
K = EPW // C
NP = 10240
RPT = NP // NS

_mesh = plsc.VectorSubcoreMesh(core_axis_name="c", subcore_axis_name="s")


@functools.partial(
    pl.kernel,
    out_type=jax.ShapeDtypeStruct((NC, NP, D), jnp.float32),
    mesh=_mesh,
    scratch_types=[
        pltpu.VMEM((K, C), jnp.int32),
        pltpu.VMEM((K, C), jnp.int32),
        pltpu.VMEM((C, D), jnp.float32),
        pltpu.VMEM_SHARED((NP, D), jnp.float32),
        pltpu.SemaphoreType.DMA,
    ],
)
def _segsum_sc(s_hbm, src_hbm, dst_hbm, zeros_hbm, out_hbm,
               src_v, dst_v, rows_v, acc_sh, sem):
    c = lax.axis_index("c")
    s = lax.axis_index("s")
    wid = s * NC + c
    pltpu.sync_copy(zeros_hbm.at[pl.ds(s * RPT, RPT)],
                    acc_sh.at[pl.ds(s * RPT, RPT)])
    pltpu.sync_copy(src_hbm.at[wid], src_v)
    pltpu.sync_copy(dst_hbm.at[wid], dst_v)
    plsc.subcore_barrier()

    def body(k, carry):
        pltpu.async_copy(s_hbm.at[src_v.at[k]], rows_v, sem).wait()
        pltpu.sync_copy(rows_v, acc_sh.at[dst_v.at[k]], add=True)
        return carry

    lax.fori_loop(0, K, body, 0, unroll=False)
    plsc.subcore_barrier()
    pltpu.sync_copy(acc_sh.at[pl.ds(s * RPT, RPT)],
                    out_hbm.at[c, pl.ds(s * RPT, RPT)])


_R = 1000


def _dot(a, b):
    return lax.dot_general(a, b, (((1,), (0,)), ((), ())),
                           preferred_element_type=jnp.float32,
                           precision=lax.Precision.HIGHEST)


def _prep_body(x_ref, deg_ref, wt_ref, bt_ref, w0_ref,
               x0_ref, s0_ref, dinv_ref):
    xb = x_ref[...]
    deg = 1.0 + deg_ref[0, :, 0:1] + deg_ref[1, :, 0:1]
    dinv = lax.rsqrt(deg)
    x0_ref[...] = _dot(xb, wt_ref[...]) + bt_ref[...]
    s0_ref[...] = _dot(xb, w0_ref[...]) * dinv
    dinv_ref[...] = jnp.broadcast_to(dinv, (_R, D))


def _prep_tc(x, degp, wt, bt2, w0):
    return pl.pallas_call(
        _prep_body,
        grid=(N // _R,),
        in_specs=[
            pl.BlockSpec((_R, D), lambda i: (i, 0)),
            pl.BlockSpec((NC, _R, D), lambda i: (0, i, 0)),
            pl.BlockSpec((D, D), lambda i: (0, 0)),
            pl.BlockSpec((1, D), lambda i: (0, 0)),
            pl.BlockSpec((D, D), lambda i: (0, 0)),
        ],
        out_specs=[
            pl.BlockSpec((_R, D), lambda i: (i, 0)),
            pl.BlockSpec((_R, D), lambda i: (i, 0)),
            pl.BlockSpec((_R, D), lambda i: (i, 0)),
        ],
        out_shape=[
            jax.ShapeDtypeStruct((N, D), jnp.float32),
            jax.ShapeDtypeStruct((N, D), jnp.float32),
            jax.ShapeDtypeStruct((N, D), jnp.float32),
        ],
    )(x, degp, wt, bt2, w0)


def _mid_body(acc_ref, s_ref, dinv_ref, x0_ref, w_ref, out_ref):
    dinv = dinv_ref[...]
    conv = (acc_ref[0] + acc_ref[1] + s_ref[...]) * dinv
    h = jnp.maximum(conv, 0.0) + x0_ref[...]
    out_ref[...] = _dot(h, w_ref[...]) * dinv


def _mid_tc(acc, s_prev, dinv, x0, w):
    return pl.pallas_call(
        _mid_body,
        grid=(N // _R,),
        in_specs=[
            pl.BlockSpec((NC, _R, D), lambda i: (0, i, 0)),
            pl.BlockSpec((_R, D), lambda i: (i, 0)),
            pl.BlockSpec((_R, D), lambda i: (i, 0)),
            pl.BlockSpec((_R, D), lambda i: (i, 0)),
            pl.BlockSpec((D, D), lambda i: (0, 0)),
        ],
        out_specs=pl.BlockSpec((_R, D), lambda i: (i, 0)),
        out_shape=jax.ShapeDtypeStruct((N, D), jnp.float32),
    )(acc, s_prev, dinv, x0, w)


def _final_body(acc_ref, s_ref, dinv_ref, out_ref):
    out_ref[...] = (acc_ref[0] + acc_ref[1] + s_ref[...]) * dinv_ref[...]


def _final_tc(acc, s2, dinv):
    return pl.pallas_call(
        _final_body,
        grid=(N // _R,),
        in_specs=[
            pl.BlockSpec((NC, _R, D), lambda i: (0, i, 0)),
            pl.BlockSpec((_R, D), lambda i: (i, 0)),
            pl.BlockSpec((_R, D), lambda i: (i, 0)),
        ],
        out_specs=pl.BlockSpec((_R, D), lambda i: (i, 0)),
        out_shape=jax.ShapeDtypeStruct((N, D), jnp.float32),
    )(acc, s2, dinv)


def kernel(x, edge_index, Wt, bt, W0, W1, W2):
    src3 = edge_index[0].astype(jnp.int32).reshape(NW, K, C)
    dst3 = edge_index[1].astype(jnp.int32).reshape(NW, K, C)
    zeros128 = jnp.zeros((NP, D), jnp.float32)
    ones128 = jnp.ones((N, D), jnp.float32)
    bt2 = bt.reshape(1, D)

    degp = _segsum_sc(ones128, src3, dst3, zeros128)
    x0, s0, dinv = _prep_tc(x, degp, Wt, bt2, W0)
    acc0 = _segsum_sc(s0, src3, dst3, zeros128)
    s1 = _mid_tc(acc0, s0, dinv, x0, W1)
    acc1 = _segsum_sc(s1, src3, dst3, zeros128)
    s2 = _mid_tc(acc1, s1, dinv, x0, W2)
    acc2 = _segsum_sc(s2, src3, dst3, zeros128)
    return _final_tc(acc2, s2, dinv)


# speedup vs baseline: 18.6879x; 1.4302x over previous
import functools

import jax
import jax.numpy as jnp
from jax import lax
from jax.experimental import pallas as pl
from jax.experimental.pallas import tpu as pltpu
from jax.experimental.pallas import tpu_sc as plsc

N = 10000
D = 128
E = 320000
NC = 2
NS = 16
NW = NC * NS
EPW = E // NW
C = 80
K = EPW // C
NP = 10240
RPT = NP // NS
SHIFT = 14

_mesh = plsc.VectorSubcoreMesh(core_axis_name="c", subcore_axis_name="s")


@functools.partial(
    pl.kernel,
    out_type=jax.ShapeDtypeStruct((NC, NP, D), jnp.float32),
    mesh=_mesh,
    scratch_types=[
        pltpu.VMEM((K, C), jnp.int32),
        pltpu.VMEM((2, C), jnp.int32),
        pltpu.VMEM((2, C), jnp.int32),
        pltpu.VMEM((2, C, D), jnp.float32),
        pltpu.VMEM_SHARED((NP, D), jnp.float32),
        pltpu.SemaphoreType.DMA,
        pltpu.SemaphoreType.DMA,
    ],
)
def _segsum_sc(s_hbm, packed_hbm, zeros_hbm, out_hbm,
               packed_v, src_c, dst_c, rows2, acc_sh, sem_g, sem_s):
    c = lax.axis_index("c")
    s = lax.axis_index("s")
    wid = s * NC + c
    pltpu.sync_copy(zeros_hbm.at[pl.ds(s * RPT, RPT)],
                    acc_sh.at[pl.ds(s * RPT, RPT)])
    pltpu.sync_copy(packed_hbm.at[wid], packed_v)

    def unpack(k, slot, want_src, want_dst):
        for j in range(C // 16):
            p = packed_v[k, pl.ds(j * 16, 16)]
            if want_src:
                src_c[slot, pl.ds(j * 16, 16)] = jnp.bitwise_and(
                    p, (1 << SHIFT) - 1)
            if want_dst:
                dst_c[slot, pl.ds(j * 16, 16)] = lax.shift_right_logical(
                    p, SHIFT)

    plsc.subcore_barrier()

    unpack(0, 0, True, False)
    pltpu.async_copy(s_hbm.at[src_c.at[0]], rows2.at[0], sem_g)

    def body(k, carry):
        b = lax.rem(k, 2)
        pltpu.make_async_copy(s_hbm.at[pl.ds(0, C)], rows2.at[0],
                              sem_g).wait()

        @pl.when(k >= 2)
        def _():
            pltpu.make_async_copy(s_hbm.at[pl.ds(0, C)], rows2.at[0],
                                  sem_s).wait()

        unpack(k, b, False, True)

        @pl.when(k + 1 < K)
        def _():
            unpack(k + 1, 1 - b, True, False)
            pltpu.async_copy(s_hbm.at[src_c.at[1 - b]], rows2.at[1 - b],
                             sem_g)

        pltpu.async_copy(rows2.at[b], acc_sh.at[dst_c.at[b]], sem_s,
                         add=True)
        return carry

    lax.fori_loop(0, K, body, 0, unroll=False)
    pltpu.make_async_copy(s_hbm.at[pl.ds(0, C)], rows2.at[0], sem_s).wait()
    pltpu.make_async_copy(s_hbm.at[pl.ds(0, C)], rows2.at[0], sem_s).wait()
    plsc.subcore_barrier()
    pltpu.sync_copy(acc_sh.at[pl.ds(s * RPT, RPT)],
                    out_hbm.at[c, pl.ds(s * RPT, RPT)])


@functools.partial(
    pl.kernel,
    out_type=jax.ShapeDtypeStruct((NC, NP, D), jnp.float32),
    mesh=_mesh,
    scratch_types=[
        pltpu.VMEM((K, C), jnp.int32),
        pltpu.VMEM((C, D), jnp.float32),
        pltpu.VMEM_SHARED((NP, D), jnp.float32),
        pltpu.SemaphoreType.DMA,
    ],
)
def _deg_sc(ones_hbm, dst_hbm, zeros_hbm, out_hbm,
            dst_v, ones_v, acc_sh, sem_s):
    c = lax.axis_index("c")
    s = lax.axis_index("s")
    wid = s * NC + c
    pltpu.sync_copy(zeros_hbm.at[pl.ds(s * RPT, RPT)],
                    acc_sh.at[pl.ds(s * RPT, RPT)])
    pltpu.sync_copy(dst_hbm.at[wid], dst_v)
    pltpu.sync_copy(ones_hbm, ones_v)
    plsc.subcore_barrier()

    def body(k, carry):
        @pl.when(k >= 2)
        def _():
            pltpu.make_async_copy(ones_hbm, ones_v, sem_s).wait()

        pltpu.async_copy(ones_v, acc_sh.at[dst_v.at[k]], sem_s, add=True)
        return carry

    lax.fori_loop(0, K, body, 0, unroll=False)
    pltpu.make_async_copy(ones_hbm, ones_v, sem_s).wait()
    pltpu.make_async_copy(ones_hbm, ones_v, sem_s).wait()
    plsc.subcore_barrier()
    pltpu.sync_copy(acc_sh.at[pl.ds(s * RPT, RPT)],
                    out_hbm.at[c, pl.ds(s * RPT, RPT)])


_R = 1000


def _dot(a, b):
    return lax.dot_general(a, b, (((1,), (0,)), ((), ())),
                           preferred_element_type=jnp.float32,
                           precision=lax.Precision.HIGHEST)


def _prep_body(x_ref, deg_ref, wt_ref, bt_ref, w0_ref,
               x0_ref, s0_ref, dinv_ref):
    xb = x_ref[...]
    deg = 1.0 + deg_ref[0, :, 0:1] + deg_ref[1, :, 0:1]
    dinv = lax.rsqrt(deg)
    x0_ref[...] = _dot(xb, wt_ref[...]) + bt_ref[...]
    s0_ref[...] = _dot(xb, w0_ref[...]) * dinv
    dinv_ref[...] = jnp.broadcast_to(dinv, (_R, D))


def _prep_tc(x, degp, wt, bt2, w0):
    return pl.pallas_call(
        _prep_body,
        grid=(N // _R,),
        in_specs=[
            pl.BlockSpec((_R, D), lambda i: (i, 0)),
            pl.BlockSpec((NC, _R, D), lambda i: (0, i, 0)),
            pl.BlockSpec((D, D), lambda i: (0, 0)),
            pl.BlockSpec((1, D), lambda i: (0, 0)),
            pl.BlockSpec((D, D), lambda i: (0, 0)),
        ],
        out_specs=[
            pl.BlockSpec((_R, D), lambda i: (i, 0)),
            pl.BlockSpec((_R, D), lambda i: (i, 0)),
            pl.BlockSpec((_R, D), lambda i: (i, 0)),
        ],
        out_shape=[
            jax.ShapeDtypeStruct((N, D), jnp.float32),
            jax.ShapeDtypeStruct((N, D), jnp.float32),
            jax.ShapeDtypeStruct((N, D), jnp.float32),
        ],
    )(x, degp, wt, bt2, w0)


def _mid_body(acc_ref, s_ref, dinv_ref, x0_ref, w_ref, out_ref):
    dinv = dinv_ref[...]
    conv = (acc_ref[0] + acc_ref[1] + s_ref[...]) * dinv
    h = jnp.maximum(conv, 0.0) + x0_ref[...]
    out_ref[...] = _dot(h, w_ref[...]) * dinv


def _mid_tc(acc, s_prev, dinv, x0, w):
    return pl.pallas_call(
        _mid_body,
        grid=(N // _R,),
        in_specs=[
            pl.BlockSpec((NC, _R, D), lambda i: (0, i, 0)),
            pl.BlockSpec((_R, D), lambda i: (i, 0)),
            pl.BlockSpec((_R, D), lambda i: (i, 0)),
            pl.BlockSpec((_R, D), lambda i: (i, 0)),
            pl.BlockSpec((D, D), lambda i: (0, 0)),
        ],
        out_specs=pl.BlockSpec((_R, D), lambda i: (i, 0)),
        out_shape=jax.ShapeDtypeStruct((N, D), jnp.float32),
    )(acc, s_prev, dinv, x0, w)


def _final_body(acc_ref, s_ref, dinv_ref, out_ref):
    out_ref[...] = (acc_ref[0] + acc_ref[1] + s_ref[...]) * dinv_ref[...]


def _final_tc(acc, s2, dinv):
    return pl.pallas_call(
        _final_body,
        grid=(N // _R,),
        in_specs=[
            pl.BlockSpec((NC, _R, D), lambda i: (0, i, 0)),
            pl.BlockSpec((_R, D), lambda i: (i, 0)),
            pl.BlockSpec((_R, D), lambda i: (i, 0)),
        ],
        out_specs=pl.BlockSpec((_R, D), lambda i: (i, 0)),
        out_shape=jax.ShapeDtypeStruct((N, D), jnp.float32),
    )(acc, s2, dinv)


def kernel(x, edge_index, Wt, bt, W0, W1, W2):
    src = edge_index[0].astype(jnp.int32)
    dst = edge_index[1].astype(jnp.int32)
    packed3 = (src | (dst << SHIFT)).reshape(NW, K, C)
    dst3 = dst.reshape(NW, K, C)
    zeros128 = jnp.zeros((NP, D), jnp.float32)
    ones_c = jnp.ones((C, D), jnp.float32)
    bt2 = bt.reshape(1, D)

    degp = _deg_sc(ones_c, dst3, zeros128)
    x0, s0, dinv = _prep_tc(x, degp, Wt, bt2, W0)
    acc0 = _segsum_sc(s0, packed3, zeros128)
    s1 = _mid_tc(acc0, s0, dinv, x0, W1)
    acc1 = _segsum_sc(s1, packed3, zeros128)
    s2 = _mid_tc(acc1, s1, dinv, x0, W2)
    acc2 = _segsum_sc(s2, packed3, zeros128)
    return _final_tc(acc2, s2, dinv)


# speedup vs baseline: 25.9351x; 1.3878x over previous
import functools

import jax
import jax.numpy as jnp
from jax import lax
from jax.experimental import pallas as pl
from jax.experimental.pallas import tpu as pltpu
from jax.experimental.pallas import tpu_sc as plsc

N = 10000
D = 128
E = 320000
NC = 2
NS = 16
NW = NC * NS
EPW = E // NW
C = 80
K = EPW // C
KB = 25
NB = K // KB
NP = 10240
RPT = NP // NS
SHIFT = 14

_mesh = plsc.VectorSubcoreMesh(core_axis_name="c", subcore_axis_name="s")


@functools.partial(
    pl.kernel,
    out_type=jax.ShapeDtypeStruct((NC, NP, D), jnp.float32),
    mesh=_mesh,
    scratch_types=[
        pltpu.VMEM((2, KB, C), jnp.int32),
        pltpu.VMEM((3, C), jnp.int32),
        pltpu.VMEM((3, C), jnp.int32),
        pltpu.VMEM((3, C, D), jnp.float32),
        pltpu.VMEM_SHARED((NP, D), jnp.float32),
        pltpu.SemaphoreType.DMA,
        pltpu.SemaphoreType.DMA,
        pltpu.SemaphoreType.DMA,
    ],
)
def _segsum_sc(s_hbm, packed_hbm, zeros_hbm, out_hbm,
               packed2, src_c, dst_c, rows3, acc_sh, sem_g, sem_s, sem_p):
    c = lax.axis_index("c")
    s = lax.axis_index("s")
    wid = s * NC + c
    pltpu.sync_copy(zeros_hbm.at[pl.ds(s * RPT, RPT)],
                    acc_sh.at[pl.ds(s * RPT, RPT)])
    pltpu.sync_copy(packed_hbm.at[wid, 0], packed2.at[0])
    pltpu.async_copy(packed_hbm.at[wid, 1], packed2.at[1], sem_p)

    def unpack(slot, r, k3, want_src, want_dst):
        for j in range(C // 16):
            p = packed2[slot, r, pl.ds(j * 16, 16)]
            if want_src:
                src_c[k3, pl.ds(j * 16, 16)] = jnp.bitwise_and(
                    p, (1 << SHIFT) - 1)
            if want_dst:
                dst_c[k3, pl.ds(j * 16, 16)] = lax.shift_right_logical(
                    p, SHIFT)

    plsc.subcore_barrier()

    unpack(0, 0, 0, True, False)
    unpack(0, 1, 1, True, False)
    pltpu.async_copy(s_hbm.at[src_c.at[0]], rows3.at[0], sem_g)
    pltpu.async_copy(s_hbm.at[src_c.at[1]], rows3.at[1], sem_g)

    def body(k, carry):
        bg = lax.rem(k, 3)
        m = lax.div(k, KB)
        r = lax.rem(k, KB)
        slot = lax.rem(m, 2)
        pltpu.make_async_copy(s_hbm.at[pl.ds(0, C)], rows3.at[0],
                              sem_g).wait()

        @pl.when(k >= 1)
        def _():
            pltpu.make_async_copy(s_hbm.at[pl.ds(0, C)], rows3.at[0],
                                  sem_s).wait()

        @pl.when(jnp.logical_and(r == 0, jnp.logical_and(m >= 1,
                                                         m + 1 < NB)))
        def _():
            pltpu.async_copy(packed_hbm.at[wid, m + 1], packed2.at[1 - slot],
                             sem_p)

        @pl.when(jnp.logical_and(r == KB - 2, k + 2 < K))
        def _():
            pltpu.make_async_copy(packed_hbm.at[wid, 0], packed2.at[0],
                                  sem_p).wait()

        unpack(slot, r, bg, False, True)

        @pl.when(k + 2 < K)
        def _():
            k2 = k + 2
            m2 = lax.div(k2, KB)
            unpack(lax.rem(m2, 2), lax.rem(k2, KB), lax.rem(k2, 3),
                   True, False)
            pltpu.async_copy(s_hbm.at[src_c.at[lax.rem(k2, 3)]],
                             rows3.at[lax.rem(k2, 3)], sem_g)

        pltpu.async_copy(rows3.at[bg], acc_sh.at[dst_c.at[bg]], sem_s,
                         add=True)
        return carry

    lax.fori_loop(0, K, body, 0, unroll=False)
    pltpu.make_async_copy(s_hbm.at[pl.ds(0, C)], rows3.at[0], sem_s).wait()
    plsc.subcore_barrier()
    pltpu.sync_copy(acc_sh.at[pl.ds(s * RPT, RPT)],
                    out_hbm.at[c, pl.ds(s * RPT, RPT)])


@functools.partial(
    pl.kernel,
    out_type=jax.ShapeDtypeStruct((NC, NP, D), jnp.float32),
    mesh=_mesh,
    scratch_types=[
        pltpu.VMEM((K, C), jnp.int32),
        pltpu.VMEM((C, D), jnp.float32),
        pltpu.VMEM_SHARED((NP, D), jnp.float32),
        pltpu.SemaphoreType.DMA,
    ],
)
def _deg_sc(ones_hbm, dst_hbm, zeros_hbm, out_hbm,
            dst_v, ones_v, acc_sh, sem_s):
    c = lax.axis_index("c")
    s = lax.axis_index("s")
    wid = s * NC + c
    pltpu.sync_copy(zeros_hbm.at[pl.ds(s * RPT, RPT)],
                    acc_sh.at[pl.ds(s * RPT, RPT)])
    pltpu.sync_copy(dst_hbm.at[wid], dst_v)
    pltpu.sync_copy(ones_hbm, ones_v)
    plsc.subcore_barrier()

    def body(k, carry):
        @pl.when(k >= 2)
        def _():
            pltpu.make_async_copy(ones_hbm, ones_v, sem_s).wait()

        pltpu.async_copy(ones_v, acc_sh.at[dst_v.at[k]], sem_s, add=True)
        return carry

    lax.fori_loop(0, K, body, 0, unroll=False)
    pltpu.make_async_copy(ones_hbm, ones_v, sem_s).wait()
    pltpu.make_async_copy(ones_hbm, ones_v, sem_s).wait()
    plsc.subcore_barrier()
    pltpu.sync_copy(acc_sh.at[pl.ds(s * RPT, RPT)],
                    out_hbm.at[c, pl.ds(s * RPT, RPT)])


_R = 1000


def _dot(a, b):
    return lax.dot_general(a, b, (((1,), (0,)), ((), ())),
                           preferred_element_type=jnp.float32,
                           precision=lax.Precision.HIGHEST)


def _prep_body(x_ref, deg_ref, wt_ref, bt_ref, w0_ref,
               x0_ref, s0_ref, dinv_ref):
    xb = x_ref[...]
    deg = 1.0 + deg_ref[0, :, 0:1] + deg_ref[1, :, 0:1]
    dinv = lax.rsqrt(deg)
    x0_ref[...] = _dot(xb, wt_ref[...]) + bt_ref[...]
    s0_ref[...] = _dot(xb, w0_ref[...]) * dinv
    dinv_ref[...] = jnp.broadcast_to(dinv, (_R, D))


def _prep_tc(x, degp, wt, bt2, w0):
    return pl.pallas_call(
        _prep_body,
        grid=(N // _R,),
        in_specs=[
            pl.BlockSpec((_R, D), lambda i: (i, 0)),
            pl.BlockSpec((NC, _R, D), lambda i: (0, i, 0)),
            pl.BlockSpec((D, D), lambda i: (0, 0)),
            pl.BlockSpec((1, D), lambda i: (0, 0)),
            pl.BlockSpec((D, D), lambda i: (0, 0)),
        ],
        out_specs=[
            pl.BlockSpec((_R, D), lambda i: (i, 0)),
            pl.BlockSpec((_R, D), lambda i: (i, 0)),
            pl.BlockSpec((_R, D), lambda i: (i, 0)),
        ],
        out_shape=[
            jax.ShapeDtypeStruct((N, D), jnp.float32),
            jax.ShapeDtypeStruct((N, D), jnp.float32),
            jax.ShapeDtypeStruct((N, D), jnp.float32),
        ],
    )(x, degp, wt, bt2, w0)


def _mid_body(acc_ref, s_ref, dinv_ref, x0_ref, w_ref, out_ref):
    dinv = dinv_ref[...]
    conv = (acc_ref[0] + acc_ref[1] + s_ref[...]) * dinv
    h = jnp.maximum(conv, 0.0) + x0_ref[...]
    out_ref[...] = _dot(h, w_ref[...]) * dinv


def _mid_tc(acc, s_prev, dinv, x0, w):
    return pl.pallas_call(
        _mid_body,
        grid=(N // _R,),
        in_specs=[
            pl.BlockSpec((NC, _R, D), lambda i: (0, i, 0)),
            pl.BlockSpec((_R, D), lambda i: (i, 0)),
            pl.BlockSpec((_R, D), lambda i: (i, 0)),
            pl.BlockSpec((_R, D), lambda i: (i, 0)),
            pl.BlockSpec((D, D), lambda i: (0, 0)),
        ],
        out_specs=pl.BlockSpec((_R, D), lambda i: (i, 0)),
        out_shape=jax.ShapeDtypeStruct((N, D), jnp.float32),
    )(acc, s_prev, dinv, x0, w)


def _final_body(acc_ref, s_ref, dinv_ref, out_ref):
    out_ref[...] = (acc_ref[0] + acc_ref[1] + s_ref[...]) * dinv_ref[...]


def _final_tc(acc, s2, dinv):
    return pl.pallas_call(
        _final_body,
        grid=(N // _R,),
        in_specs=[
            pl.BlockSpec((NC, _R, D), lambda i: (0, i, 0)),
            pl.BlockSpec((_R, D), lambda i: (i, 0)),
            pl.BlockSpec((_R, D), lambda i: (i, 0)),
        ],
        out_specs=pl.BlockSpec((_R, D), lambda i: (i, 0)),
        out_shape=jax.ShapeDtypeStruct((N, D), jnp.float32),
    )(acc, s2, dinv)


def kernel(x, edge_index, Wt, bt, W0, W1, W2):
    src = edge_index[0].astype(jnp.int32)
    dst = edge_index[1].astype(jnp.int32)
    packed3 = (src | (dst << SHIFT)).reshape(NW, NB, KB, C)
    dst3 = dst.reshape(NW, K, C)
    zeros128 = jnp.zeros((NP, D), jnp.float32)
    ones_c = jnp.ones((C, D), jnp.float32)
    bt2 = bt.reshape(1, D)

    degp = _deg_sc(ones_c, dst3, zeros128)
    x0, s0, dinv = _prep_tc(x, degp, Wt, bt2, W0)
    acc0 = _segsum_sc(s0, packed3, zeros128)
    s1 = _mid_tc(acc0, s0, dinv, x0, W1)
    acc1 = _segsum_sc(s1, packed3, zeros128)
    s2 = _mid_tc(acc1, s1, dinv, x0, W2)
    acc2 = _segsum_sc(s2, packed3, zeros128)
    return _final_tc(acc2, s2, dinv)
